# baseline (speedup 1.0000x reference)
import math
import functools

import jax
import jax.numpy as jnp
from jax.experimental import pallas as pl

_DIM = 1024
_NEG_LOG_FREQ_OVER_DIM = -math.log(10000.0) / _DIM


def _pe_add_block(emb_ref, out_ref, *, s_blk, max_len):
    base = (pl.program_id(0) * s_blk) % max_len
    row = jax.lax.broadcasted_iota(jnp.int32, (s_blk, _DIM), 0)
    lane = jax.lax.broadcasted_iota(jnp.int32, (s_blk, _DIM), 1)
    inv_freq = jnp.exp((lane - (lane % 2)).astype(jnp.float32)
                       * _NEG_LOG_FREQ_OVER_DIM)
    ang = (row + base).astype(jnp.float32) * inv_freq
    pe = jnp.where(lane % 2 == 0, jnp.sin(ang), jnp.cos(ang))
    out_ref[...] = emb_ref[...] + pe


@jax.jit
def kernel(position_ids, embeddings):
    batch, max_len, dim = embeddings.shape
    s_blk = 2048
    flat = embeddings.reshape(batch * max_len, dim)
    grid = (flat.shape[0] // s_blk,)
    out = pl.pallas_call(
        functools.partial(_pe_add_block, s_blk=s_blk, max_len=max_len),
        grid=grid,
        in_specs=[pl.BlockSpec((s_blk, dim), lambda i: (i, 0))],
        out_specs=pl.BlockSpec((s_blk, dim), lambda i: (i, 0)),
        out_shape=jax.ShapeDtypeStruct(flat.shape, flat.dtype),
    )(flat)
    return out.reshape(batch, max_len, dim)


# speedup vs baseline: 5.0911x; 5.0911x over previous
import math
import functools

import jax
import jax.numpy as jnp
from jax.experimental import pallas as pl
from jax.experimental.pallas import tpu as pltpu

_DIM = 1024
_NEG_LOG_FREQ_OVER_DIM = -math.log(10000.0) / _DIM
_SUB = 256


def _pe_add_block(emb_ref, out_ref, sr_ref, cr_ref, *, s_blk, max_len):
    i = pl.program_id(0)
    lane1 = jax.lax.broadcasted_iota(jnp.int32, (1, _DIM), 1)
    even1 = (lane1 % 2) == 0
    inv_freq1 = jnp.exp((lane1 - (lane1 % 2)).astype(jnp.float32)
                        * _NEG_LOG_FREQ_OVER_DIM)

    @pl.when(i == 0)
    def _init_scratch():
        row = jax.lax.broadcasted_iota(jnp.int32, (_SUB, _DIM), 0)
        lane = jax.lax.broadcasted_iota(jnp.int32, (_SUB, _DIM), 1)
        inv_freq = jnp.exp((lane - (lane % 2)).astype(jnp.float32)
                           * _NEG_LOG_FREQ_OVER_DIM)
        r_ang = row.astype(jnp.float32) * inv_freq
        sr_ref[...] = jnp.sin(r_ang)
        cr_ref[...] = jnp.cos(r_ang)

    block_base = (i * s_blk) % max_len
    sr = sr_ref[...]
    cr = cr_ref[...]
    for a in range(s_blk // _SUB):
        b_ang = (block_base + a * _SUB).astype(jnp.float32) * inv_freq1
        sb = jnp.sin(b_ang)
        cb = jnp.cos(b_ang)
        coeff_a = jnp.where(even1, cb, -sb)
        coeff_b = jnp.where(even1, sb, cb)
        sl = pl.ds(a * _SUB, _SUB)
        out_ref[sl, :] = emb_ref[sl, :] + (sr * coeff_a + cr * coeff_b)


@jax.jit
def kernel(position_ids, embeddings):
    batch, max_len, dim = embeddings.shape
    s_blk = 2048
    flat = embeddings.reshape(batch * max_len, dim)
    grid = (flat.shape[0] // s_blk,)
    out = pl.pallas_call(
        functools.partial(_pe_add_block, s_blk=s_blk, max_len=max_len),
        grid=grid,
        in_specs=[pl.BlockSpec((s_blk, dim), lambda i: (i, 0))],
        out_specs=pl.BlockSpec((s_blk, dim), lambda i: (i, 0)),
        out_shape=jax.ShapeDtypeStruct(flat.shape, flat.dtype),
        scratch_shapes=[
            pltpu.VMEM((_SUB, _DIM), jnp.float32),
            pltpu.VMEM((_SUB, _DIM), jnp.float32),
        ],
    )(flat)
    return out.reshape(batch, max_len, dim)
